# baseline (speedup 1.0000x reference)
import functools

import jax
import jax.numpy as jnp
import numpy as np
from jax.experimental import pallas as pl
from jax.experimental.pallas import tpu as pltpu

RES = 64
UPSCALING = 1
K = 30
SPECTRUM_RES = 3
F = SPECTRUM_RES * SPECTRUM_RES


def _gabor_block_kernel(params_ref, gx_ref, gy_ref, dx_ref, dy_ref, radt_ref,
                        out_ref):
    inv2s = params_ref[0, 0]
    k = K
    x = dx_ref[...] - gx_ref[...]
    y = dy_ref[...] - gy_ref[...]
    w = jnp.exp(-(x * x + y * y) * inv2s)
    out_ref[0, :] = jnp.sum(w, axis=0) * (1.0 / k)
    two_pi = 2.0 * np.pi
    for f in range(F):
        fx = params_ref[0, 1 + f]
        fy = params_ref[0, 1 + F + f]
        pf = jnp.cos(two_pi * (fx * x + fy * y)) * w
        for a in range(8):
            c = (f * 64 + a * 9) % 72
            out_ref[1 + c, :] = jnp.sum(pf * radt_ref[a, :, :], axis=0) * (1.0 / k)


def kernel(pts, rad, kernel_sigma, grid_pts, grid_freq):
    res = RES * UPSCALING
    M = res * res
    N = pts.shape[0]
    k = K
    A = rad.shape[1]

    rad_max = jnp.max(rad, axis=0, keepdims=True)
    rad_n = rad / rad_max

    d2 = (jnp.sum(grid_pts ** 2, axis=1)[:, None]
          - 2.0 * grid_pts @ pts.T
          + jnp.sum(pts ** 2, axis=1)[None, :])
    _, idx = jax.lax.top_k(-d2, k)

    gx = pts[:, 0][idx]
    gy = pts[:, 1][idx]
    radt = jnp.transpose(rad_n[idx], (2, 1, 0))

    sigma = kernel_sigma / UPSCALING
    inv2s = 0.5 / (sigma * sigma)
    fx = grid_freq[0].reshape(F)
    fy = grid_freq[1].reshape(F)
    params = jnp.concatenate(
        [jnp.asarray(inv2s, jnp.float32).reshape(1), fx, fy]).reshape(1, 2 * F + 1)

    Mb = 512
    grid = (M // Mb,)
    out = pl.pallas_call(
        _gabor_block_kernel,
        grid=grid,
        in_specs=[
            pl.BlockSpec((1, 2 * F + 1), lambda i: (0, 0)),
            pl.BlockSpec((1, Mb), lambda i: (0, i)),
            pl.BlockSpec((1, Mb), lambda i: (0, i)),
            pl.BlockSpec((k, Mb), lambda i: (0, i)),
            pl.BlockSpec((k, Mb), lambda i: (0, i)),
            pl.BlockSpec((A, k, Mb), lambda i: (0, 0, i)),
        ],
        out_specs=pl.BlockSpec((1 + 8 * F, Mb), lambda i: (0, i)),
        out_shape=jax.ShapeDtypeStruct((1 + 8 * F, M), jnp.float32),
    )(params,
      grid_pts[:, 0].reshape(1, M),
      grid_pts[:, 1].reshape(1, M),
      jnp.transpose(gx, (1, 0)),
      jnp.transpose(gy, (1, 0)),
      radt)

    return out.reshape(1 + 8 * F, res, res).astype(jnp.float32)
